# Initial kernel scaffold; baseline (speedup 1.0000x reference)
import jax
import jax.numpy as jnp
from jax.experimental import pallas as pl


def kernel(x, pos, batch, x_skip, pos_skip, batch_skip, W, b):
    raise NotImplementedError("write your pallas kernel here")


# speedup vs baseline: 7.8086x; 7.8086x over previous
import jax
import jax.numpy as jnp
from jax.experimental import pallas as pl

K = 3
_R = 256


def _fused_body(ps3_ref, bsk_ref, p3t_ref, bt_ref, ns_ref, np_ref,
                x_ref, xs_ref, w_ref, b_ref, out_ref):
    cross = jnp.dot(ps3_ref[...], p3t_ref[...],
                    preferred_element_type=jnp.float32)
    d2 = (ns_ref[...] + np_ref[...]) - 2.0 * cross
    d2 = jnp.where(bsk_ref[...] != bt_ref[...], jnp.float32(jnp.inf), d2)

    iota = jax.lax.broadcasted_iota(jnp.int32, d2.shape, 1)
    big_i = jnp.int32(2 ** 30)
    inf = jnp.float32(jnp.inf)

    cur = d2
    s = jnp.zeros_like(d2)
    wsum = jnp.zeros((d2.shape[0], 1), jnp.float32)
    for _ in range(K):
        m = jnp.min(cur, axis=1, keepdims=True)
        idxk = jnp.min(jnp.where(cur == m, iota, big_i), axis=1, keepdims=True)
        onehot = (iota == idxk)
        wk = 1.0 / jnp.maximum(m, 1e-16)
        s = s + jnp.where(onehot, wk, 0.0)
        wsum = wsum + wk
        cur = jnp.where(onehot, inf, cur)

    interp = jnp.dot(s, x_ref[...], preferred_element_type=jnp.float32,
                     precision=jax.lax.Precision.HIGHEST) / wsum
    h = jnp.concatenate([interp, xs_ref[...]], axis=1)
    out_ref[...] = (jnp.dot(h, w_ref[...], preferred_element_type=jnp.float32)
                    + b_ref[...])


def kernel(x, pos, batch, x_skip, pos_skip, batch_skip, W, b):
    n_src, c_in = x.shape
    n_dst, c_skip = x_skip.shape
    c_out = W.shape[1]

    p3t = pos.T
    bt = batch.astype(jnp.float32)[None, :]
    bsk = batch_skip.astype(jnp.float32)[:, None]
    ns = jnp.sum(pos_skip * pos_skip, axis=-1)[:, None]
    npp = jnp.sum(pos * pos, axis=-1)[None, :]
    b2 = b[None, :]

    grid = (n_dst // _R,)
    out = pl.pallas_call(
        _fused_body,
        grid=grid,
        in_specs=[
            pl.BlockSpec((_R, 3), lambda i: (i, 0)),
            pl.BlockSpec((_R, 1), lambda i: (i, 0)),
            pl.BlockSpec((3, n_src), lambda i: (0, 0)),
            pl.BlockSpec((1, n_src), lambda i: (0, 0)),
            pl.BlockSpec((_R, 1), lambda i: (i, 0)),
            pl.BlockSpec((1, n_src), lambda i: (0, 0)),
            pl.BlockSpec((n_src, c_in), lambda i: (0, 0)),
            pl.BlockSpec((_R, c_skip), lambda i: (i, 0)),
            pl.BlockSpec((c_in + c_skip, c_out), lambda i: (0, 0)),
            pl.BlockSpec((1, c_out), lambda i: (0, 0)),
        ],
        out_specs=pl.BlockSpec((_R, c_out), lambda i: (i, 0)),
        out_shape=jax.ShapeDtypeStruct((n_dst, c_out), jnp.float32),
    )(pos_skip, bsk, p3t, bt, ns, npp, x, x_skip, W, b2)

    return (out, pos_skip, batch_skip)


# speedup vs baseline: 12.5100x; 1.6021x over previous
import jax
import jax.numpy as jnp
from jax.experimental import pallas as pl

K = 3
_R = 256


def _fused_body(ps3_ref, bsk_ref, p3t_ref, bt_ref, ns_ref, np_ref,
                x_ref, xs_ref, w_ref, b_ref, out_ref):
    cross = jnp.dot(ps3_ref[...], p3t_ref[...],
                    preferred_element_type=jnp.float32)
    d2 = (ns_ref[...] + np_ref[...]) - 2.0 * cross
    d2 = jnp.where(bsk_ref[...] != bt_ref[...], jnp.float32(jnp.inf), d2)

    iota = jax.lax.broadcasted_iota(jnp.int32, d2.shape, 1)
    big_i = jnp.int32(2 ** 30)
    inf = jnp.float32(jnp.inf)

    cur = d2
    s = jnp.zeros_like(d2)
    wsum = jnp.zeros((d2.shape[0], 1), jnp.float32)
    for _ in range(K):
        m = jnp.min(cur, axis=1, keepdims=True)
        idxk = jnp.min(jnp.where(cur == m, iota, big_i), axis=1, keepdims=True)
        onehot = (iota == idxk)
        wk = 1.0 / jnp.maximum(m, 1e-16)
        s = s + jnp.where(onehot, wk, 0.0)
        wsum = wsum + wk
        cur = jnp.where(onehot, inf, cur)

    interp = jnp.dot(s, x_ref[...], preferred_element_type=jnp.float32) / wsum
    h = jnp.concatenate([interp, xs_ref[...]], axis=1)
    out_ref[...] = (jnp.dot(h, w_ref[...], preferred_element_type=jnp.float32)
                    + b_ref[...])


def kernel(x, pos, batch, x_skip, pos_skip, batch_skip, W, b):
    n_src, c_in = x.shape
    n_dst, c_skip = x_skip.shape
    c_out = W.shape[1]

    p3t = pos.T
    bt = batch.astype(jnp.float32)[None, :]
    bsk = batch_skip.astype(jnp.float32)[:, None]
    ns = jnp.sum(pos_skip * pos_skip, axis=-1)[:, None]
    npp = jnp.sum(pos * pos, axis=-1)[None, :]
    b2 = b[None, :]

    grid = (n_dst // _R,)
    out = pl.pallas_call(
        _fused_body,
        grid=grid,
        in_specs=[
            pl.BlockSpec((_R, 3), lambda i: (i, 0)),
            pl.BlockSpec((_R, 1), lambda i: (i, 0)),
            pl.BlockSpec((3, n_src), lambda i: (0, 0)),
            pl.BlockSpec((1, n_src), lambda i: (0, 0)),
            pl.BlockSpec((_R, 1), lambda i: (i, 0)),
            pl.BlockSpec((1, n_src), lambda i: (0, 0)),
            pl.BlockSpec((n_src, c_in), lambda i: (0, 0)),
            pl.BlockSpec((_R, c_skip), lambda i: (i, 0)),
            pl.BlockSpec((c_in + c_skip, c_out), lambda i: (0, 0)),
            pl.BlockSpec((1, c_out), lambda i: (0, 0)),
        ],
        out_specs=pl.BlockSpec((_R, c_out), lambda i: (i, 0)),
        out_shape=jax.ShapeDtypeStruct((n_dst, c_out), jnp.float32),
    )(pos_skip, bsk, p3t, bt, ns, npp, x, x_skip, W, b2)

    return (out, pos_skip, batch_skip)


# speedup vs baseline: 17.2493x; 1.3788x over previous
import jax
import jax.numpy as jnp
from jax.experimental import pallas as pl

K = 3
_R = 256


def _body(psm2_r, bsk_r, p3t_r, bt_r, ns_r, npp_r, x_r, xs_r, w_r, b_r,
          out_r):
    cross2 = jnp.dot(psm2_r[...], p3t_r[...],
                     preferred_element_type=jnp.float32)
    d2 = (ns_r[...] + npp_r[...]) + cross2
    d2 = jnp.where(bsk_r[...] != bt_r[...], jnp.float32(jnp.inf), d2)

    inf = jnp.float32(jnp.inf)
    cur = d2
    s = jnp.zeros_like(d2)
    wsum = jnp.zeros((d2.shape[0], 1), jnp.float32)
    for _ in range(K):
        m = jnp.min(cur, axis=1, keepdims=True)
        onehot = cur == m
        wk = 1.0 / jnp.maximum(m, 1e-16)
        s = jnp.where(onehot, wk, s)
        wsum = wsum + wk
        cur = jnp.where(onehot, inf, cur)

    interp = jnp.dot(s, x_r[...], preferred_element_type=jnp.float32) / wsum
    h = jnp.concatenate([interp, xs_r[...]], axis=1)
    out_r[...] = (jnp.dot(h, w_r[...], preferred_element_type=jnp.float32)
                  + b_r[...])


def kernel(x, pos, batch, x_skip, pos_skip, batch_skip, W, b):
    n_src, c_in = x.shape
    n_dst, c_skip = x_skip.shape
    c_out = W.shape[1]

    p3t = pos.T
    psm2 = pos_skip * (-2.0)
    bt = batch.astype(jnp.float32)[None, :]
    bsk = batch_skip.astype(jnp.float32)[:, None]
    ns = jnp.sum(pos_skip * pos_skip, axis=-1)[:, None]
    npp = jnp.sum(pos * pos, axis=-1)[None, :]
    b2 = b[None, :]
    xb = x.astype(jnp.bfloat16)

    grid = (n_dst // _R,)
    out = pl.pallas_call(
        _body,
        grid=grid,
        in_specs=[
            pl.BlockSpec((_R, 3), lambda i: (i, 0)),
            pl.BlockSpec((_R, 1), lambda i: (i, 0)),
            pl.BlockSpec((3, n_src), lambda i: (0, 0)),
            pl.BlockSpec((1, n_src), lambda i: (0, 0)),
            pl.BlockSpec((_R, 1), lambda i: (i, 0)),
            pl.BlockSpec((1, n_src), lambda i: (0, 0)),
            pl.BlockSpec((n_src, c_in), lambda i: (0, 0)),
            pl.BlockSpec((_R, c_skip), lambda i: (i, 0)),
            pl.BlockSpec((c_in + c_skip, c_out), lambda i: (0, 0)),
            pl.BlockSpec((1, c_out), lambda i: (0, 0)),
        ],
        out_specs=pl.BlockSpec((_R, c_out), lambda i: (i, 0)),
        out_shape=jax.ShapeDtypeStruct((n_dst, c_out), jnp.float32),
    )(psm2, bsk, p3t, bt, ns, npp, xb, x_skip, W, b2)

    return (out, pos_skip, batch_skip)
